# baseline (speedup 1.0000x reference)
import functools

import jax
import jax.numpy as jnp
from jax import lax
from jax.experimental import pallas as pl
from jax.experimental.pallas import tpu as pltpu
from jax.experimental.pallas import tpu_sc as plsc

L1, L2, ORBIT, DIM = 6, 4, 1, 2
NSITES = L1 * L2 * ORBIT
BATCH = 16384

NUM_CORES = 2
NUM_SUBCORES = 16
NUM_WORKERS = NUM_CORES * NUM_SUBCORES
LANES = 16
BW = BATCH // NUM_WORKERS
NCHUNK = BW // LANES
NSTREAM = BW // 128


def _sc_kernel(xt_hbm, wave_hbm, out_hbm, xv, idxv, outv, sem):
    wid = lax.axis_index("s") * NUM_CORES + lax.axis_index("c")

    pltpu.sync_copy(xt_hbm.at[:, pl.ds(wid * BW, BW)], xv)

    def chunk(c, carry):
        off = c * LANES
        acc = xv[0, pl.ds(off, LANES)]
        for i in range(1, NSITES):
            acc = acc + acc + xv[i, pl.ds(off, LANES)]
        idxv[pl.ds(off, LANES)] = acc
        return carry

    lax.fori_loop(0, NCHUNK, chunk, 0)

    copies = [
        pltpu.async_copy(
            wave_hbm.at[idxv.at[pl.ds(j * 128, 128)]], outv.at[j], sem
        )
        for j in range(NSTREAM)
    ]
    for c in copies:
        c.wait()

    pltpu.sync_copy(outv, out_hbm.at[pl.ds(wid * NSTREAM, NSTREAM)])


@jax.jit
def _run(x, wave):
    xt = jnp.transpose(x, (1, 2, 3, 0)).reshape(NSITES, BATCH)

    mesh = plsc.VectorSubcoreMesh(core_axis_name="c", subcore_axis_name="s")
    grid = functools.partial(
        pl.kernel,
        out_type=jax.ShapeDtypeStruct((BATCH // 128, 128), jnp.float32),
        mesh=mesh,
        scratch_types=[
            pltpu.VMEM((NSITES, BW), jnp.int32),
            pltpu.VMEM((BW,), jnp.int32),
            pltpu.VMEM((NSTREAM, 128), jnp.float32),
            pltpu.SemaphoreType.DMA,
        ],
        compiler_params=pltpu.CompilerParams(use_tc_tiling_on_sc=False),
    )
    return grid(_sc_kernel)(xt, wave)


def kernel(x, wave):
    return _run(x.astype(jnp.int32), wave).reshape(x.shape[:-3])
